# baseline (speedup 1.0000x reference)
import functools

import jax
import jax.numpy as jnp
from jax import lax
from jax.experimental import pallas as pl
from jax.experimental.pallas import tpu as pltpu
from jax.experimental.pallas import tpu_sc as plsc

_HEADS = 4
_DIM = 32
_NUM_NODES = 10000
_SEG = _HEADS * _NUM_NODES
_SEG_PAD = 40960
_SLC = _SEG_PAD // 16

_CHUNK = 1024


def _mesh():
    return plsc.VectorSubcoreMesh(
        core_axis_name="c", subcore_axis_name="s", num_cores=2, num_subcores=16
    )


def _sc_dense_scatter(x_i, x_j, idx2, w_flat):
    n = x_i.shape[0]
    rows = n // 128
    eph = n // _HEADS
    nchunks = n // _CHUNK
    base_trips = nchunks // 32
    extra = nchunks % 32

    quarter = 256

    @functools.partial(
        pl.kernel,
        out_type=(
            jax.ShapeDtypeStruct((n,), jnp.float32),
            jax.ShapeDtypeStruct((2 * _SEG_PAD,), jnp.float32),
        ),
        mesh=_mesh(),
        compiler_params=pltpu.CompilerParams(needs_layout_passes=False),
        scratch_types=[
            pltpu.VMEM((quarter, _DIM), jnp.float32),
            pltpu.VMEM((quarter, _DIM), jnp.float32),
            pltpu.VMEM((8, 128), jnp.int32),
            pltpu.VMEM((_CHUNK,), jnp.float32),
            pltpu.VMEM((_SLC,), jnp.float32),
            pltpu.VMEM((128,), jnp.float32),
            pltpu.VMEM_SHARED((_SEG_PAD,), jnp.float32),
            pltpu.SemaphoreType.DMA,
        ],
    )
    def body(xi_hbm, xj_hbm, idx_hbm, w_hbm, ex_hbm, part_hbm,
             xi_v, xj_v, sidx_v, ex_v, zbuf, wv, denom_sh, sem):
        cid = lax.axis_index("c")
        sid = lax.axis_index("s")
        wid = cid * 16 + sid
        lane = lax.broadcasted_iota(jnp.int32, (16,), 0)

        pltpu.sync_copy(w_hbm, wv)

        def zbody(i, carry):
            zbuf[pl.ds(i * 16, 16)] = jnp.zeros((16,), jnp.float32)
            return carry
        lax.fori_loop(0, _SLC // 16, zbody, 0)
        pltpu.sync_copy(zbuf, denom_sh.at[pl.ds(sid * _SLC, _SLC)])
        plsc.subcore_barrier()

        trips = jnp.where(wid < extra, base_trips + 1, base_trips)

        def chunk_body(k, carry):
            c = k * 32 + wid
            e0 = c * _CHUNK
            pltpu.sync_copy(idx_hbm.at[pl.ds(c * 8, 8)], sidx_v)

            def quarter_body(q, carry1):
                eq = e0 + q * quarter
                pltpu.sync_copy(xi_hbm.at[pl.ds(eq, quarter)], xi_v)
                pltpu.sync_copy(xj_hbm.at[pl.ds(eq, quarter)], xj_v)

                def group_body(g, carry2):
                    head = (eq + g * 16) // eph
                    w0 = wv[pl.ds(head * _DIM, 16)]
                    w1 = wv[pl.ds(head * _DIM + 16, 16)]
                    acc = jnp.zeros((16,), jnp.float32)
                    for e in range(16):
                        row = g * 16 + e
                        q16 = (xi_v[row, pl.ds(0, 16)]
                               * xj_v[row, pl.ds(0, 16)] * w0
                               + xi_v[row, pl.ds(16, 16)]
                               * xj_v[row, pl.ds(16, 16)] * w1)
                        acc = jnp.where(lane == e, jnp.sum(q16), acc)
                    ex_v[pl.ds(q * quarter + g * 16, 16)] = jnp.exp(acc)
                    return carry2
                lax.fori_loop(0, quarter // 16, group_body, 0)
                return carry1
            lax.fori_loop(0, _CHUNK // quarter, quarter_body, 0)

            pltpu.sync_copy(ex_v, ex_hbm.at[pl.ds(e0, _CHUNK)])
            copies = [
                pltpu.async_copy(ex_v.at[pl.ds(j * 128, 128)],
                                 denom_sh.at[sidx_v.at[j]], sem, add=True)
                for j in range(8)
            ]
            for cp in copies:
                cp.wait()
            return carry
        lax.fori_loop(0, trips, chunk_body, 0)
        plsc.subcore_barrier()

        pltpu.sync_copy(
            denom_sh.at[pl.ds(sid * _SLC, _SLC)],
            part_hbm.at[pl.ds(cid * _SEG_PAD + sid * _SLC, _SLC)],
        )

    return body(x_i, x_j, idx2, w_flat)


def _sc_gather_div(ex, idx2, partials):
    rows = idx2.shape[0]
    grows = (rows // 32) & ~7
    gch = 8
    last_extra = rows - 31 * grows

    @functools.partial(
        pl.kernel,
        out_type=jax.ShapeDtypeStruct((rows, 128), jnp.float32),
        mesh=_mesh(),
        compiler_params=pltpu.CompilerParams(needs_layout_passes=False),
        scratch_types=[
            pltpu.VMEM((_SEG_PAD,), jnp.float32),
            pltpu.VMEM((_SLC,), jnp.float32),
            pltpu.VMEM((gch * 128,), jnp.float32),
            pltpu.VMEM((gch, 128), jnp.int32),
            pltpu.VMEM((gch, 128), jnp.float32),
        ],
    )
    def body(ex_hbm, idx_hbm, part_hbm, out_hbm,
             denom_v, tmp_v, gex_v, gidx_v, out_v):
        cid = lax.axis_index("c")
        sid = lax.axis_index("s")
        wid = cid * 16 + sid

        pltpu.sync_copy(part_hbm.at[pl.ds(0, _SEG_PAD)], denom_v)

        def merge_chunk(j, carry):
            pltpu.sync_copy(
                part_hbm.at[pl.ds(_SEG_PAD + j * _SLC, _SLC)], tmp_v)

            def madd(i, c2):
                off = i * 16
                denom_v[pl.ds(j * _SLC + off, 16)] = (
                    denom_v[pl.ds(j * _SLC + off, 16)]
                    + tmp_v[pl.ds(off, 16)])
                return c2
            lax.fori_loop(0, _SLC // 16, madd, 0)
            return carry
        lax.fori_loop(0, 16, merge_chunk, 0)

        base = wid * grows
        nch = jnp.where(wid == 31, last_extra // gch, grows // gch)

        def gbody(k, carry):
            r0 = base + k * gch
            pltpu.sync_copy(ex_hbm.at[pl.ds(r0 * 128, gch * 128)], gex_v)
            pltpu.sync_copy(idx_hbm.at[pl.ds(r0, gch)], gidx_v)
            for r in range(gch):
                for c2 in range(8):
                    i16 = gidx_v[r, pl.ds(c2 * 16, 16)]
                    x16 = gex_v[pl.ds(r * 128 + c2 * 16, 16)]
                    d16 = plsc.load_gather(denom_v, [i16])
                    out_v[r, pl.ds(c2 * 16, 16)] = x16 / (d16 + 1e-16)
            pltpu.sync_copy(out_v, out_hbm.at[pl.ds(r0, gch)])
            return carry
        lax.fori_loop(0, nch, gbody, 0)

    return body(ex, idx2, partials)


def kernel(x_i, x_j, edge_index, num_nodes, a):
    n = x_i.shape[0]
    w_flat = (a[:, 0, :_DIM] * a[:, 0, _DIM:]).reshape(_HEADS * _DIM)
    idx = edge_index[1] + (num_nodes - _NUM_NODES).astype(edge_index.dtype)
    idx2 = idx.reshape(n // 128, 128)
    ex, partials = _sc_dense_scatter(x_i, x_j, idx2, w_flat)
    out = _sc_gather_div(ex, idx2, partials)
    return out.reshape(n, 1)


# speedup vs baseline: 3.1436x; 3.1436x over previous
import functools

import jax
import jax.numpy as jnp
from jax import lax
from jax.experimental import pallas as pl
from jax.experimental.pallas import tpu as pltpu
from jax.experimental.pallas import tpu_sc as plsc

_HEADS = 4
_DIM = 32
_NUM_NODES = 10000
_SEG = _HEADS * _NUM_NODES
_SEG_PAD = 40960
_SLC = _SEG_PAD // 16

_CHUNK = 1024


def _mesh():
    return plsc.VectorSubcoreMesh(
        core_axis_name="c", subcore_axis_name="s", num_cores=2, num_subcores=16
    )


def _sc_dense_scatter(xt_i, xt_j, idx2, w_flat):
    n = xt_i.shape[1]
    rows = n // 128
    eph = n // _HEADS
    nchunks = n // _CHUNK
    base_trips = nchunks // 32
    extra = nchunks % 32

    quarter = 256

    @functools.partial(
        pl.kernel,
        out_type=(
            jax.ShapeDtypeStruct((n,), jnp.float32),
            jax.ShapeDtypeStruct((2 * _SEG_PAD,), jnp.float32),
        ),
        mesh=_mesh(),
        compiler_params=pltpu.CompilerParams(needs_layout_passes=False),
        scratch_types=[
            pltpu.VMEM((_DIM, quarter), jnp.float32),
            pltpu.VMEM((_DIM, quarter), jnp.float32),
            pltpu.VMEM((8, 128), jnp.int32),
            pltpu.VMEM((_CHUNK,), jnp.float32),
            pltpu.VMEM((_SLC,), jnp.float32),
            pltpu.VMEM((128,), jnp.float32),
            pltpu.VMEM_SHARED((_SEG_PAD,), jnp.float32),
            pltpu.SemaphoreType.DMA,
        ],
    )
    def body(xi_hbm, xj_hbm, idx_hbm, w_hbm, ex_hbm, part_hbm,
             xi_v, xj_v, sidx_v, ex_v, zbuf, wv, denom_sh, sem):
        cid = lax.axis_index("c")
        sid = lax.axis_index("s")
        wid = cid * 16 + sid

        pltpu.sync_copy(w_hbm, wv)

        def zbody(i, carry):
            zbuf[pl.ds(i * 16, 16)] = jnp.zeros((16,), jnp.float32)
            return carry
        lax.fori_loop(0, _SLC // 16, zbody, 0)
        pltpu.sync_copy(zbuf, denom_sh.at[pl.ds(sid * _SLC, _SLC)])
        plsc.subcore_barrier()

        trips = jnp.where(wid < extra, base_trips + 1, base_trips)

        def chunk_body(k, carry):
            c = k * 32 + wid
            e0 = c * _CHUNK
            pltpu.sync_copy(idx_hbm.at[pl.ds(c * 8, 8)], sidx_v)

            def quarter_body(q, carry1):
                eq = e0 + q * quarter
                ci = pltpu.async_copy(
                    xi_hbm.at[:, pl.ds(eq, quarter)], xi_v, sem)
                cj = pltpu.async_copy(
                    xj_hbm.at[:, pl.ds(eq, quarter)], xj_v, sem)
                ci.wait()
                cj.wait()

                head = eq // eph
                wbc = [
                    plsc.load_gather(
                        wv, [jnp.full((16,), head * _DIM + d, jnp.int32)])
                    for d in range(_DIM)
                ]

                def group_body(g, carry2):
                    sl = pl.ds(g * 16, 16)
                    acc = xi_v[0, sl] * xj_v[0, sl] * wbc[0]
                    for d in range(1, _DIM):
                        acc = acc + xi_v[d, sl] * xj_v[d, sl] * wbc[d]
                    ex_v[pl.ds(q * quarter + g * 16, 16)] = jnp.exp(acc)
                    return carry2
                lax.fori_loop(0, quarter // 16, group_body, 0)
                return carry1
            lax.fori_loop(0, _CHUNK // quarter, quarter_body, 0)

            pltpu.sync_copy(ex_v, ex_hbm.at[pl.ds(e0, _CHUNK)])
            copies = [
                pltpu.async_copy(ex_v.at[pl.ds(j * 128, 128)],
                                 denom_sh.at[sidx_v.at[j]], sem, add=True)
                for j in range(8)
            ]
            for cp in copies:
                cp.wait()
            return carry
        lax.fori_loop(0, trips, chunk_body, 0)
        plsc.subcore_barrier()

        pltpu.sync_copy(
            denom_sh.at[pl.ds(sid * _SLC, _SLC)],
            part_hbm.at[pl.ds(cid * _SEG_PAD + sid * _SLC, _SLC)],
        )

    return body(xt_i, xt_j, idx2, w_flat)


def _sc_gather_div(ex, idx2, partials):
    rows = idx2.shape[0]
    grows = (rows // 32) & ~7
    gch = 8
    last_extra = rows - 31 * grows

    @functools.partial(
        pl.kernel,
        out_type=jax.ShapeDtypeStruct((rows, 128), jnp.float32),
        mesh=_mesh(),
        compiler_params=pltpu.CompilerParams(needs_layout_passes=False),
        scratch_types=[
            pltpu.VMEM((_SEG_PAD,), jnp.float32),
            pltpu.VMEM((_SLC,), jnp.float32),
            pltpu.VMEM((gch * 128,), jnp.float32),
            pltpu.VMEM((gch, 128), jnp.int32),
            pltpu.VMEM((gch, 128), jnp.float32),
        ],
    )
    def body(ex_hbm, idx_hbm, part_hbm, out_hbm,
             denom_v, tmp_v, gex_v, gidx_v, out_v):
        cid = lax.axis_index("c")
        sid = lax.axis_index("s")
        wid = cid * 16 + sid

        pltpu.sync_copy(part_hbm.at[pl.ds(0, _SEG_PAD)], denom_v)

        def merge_chunk(j, carry):
            pltpu.sync_copy(
                part_hbm.at[pl.ds(_SEG_PAD + j * _SLC, _SLC)], tmp_v)

            def madd(i, c2):
                off = i * 16
                denom_v[pl.ds(j * _SLC + off, 16)] = (
                    denom_v[pl.ds(j * _SLC + off, 16)]
                    + tmp_v[pl.ds(off, 16)])
                return c2
            lax.fori_loop(0, _SLC // 16, madd, 0)
            return carry
        lax.fori_loop(0, 16, merge_chunk, 0)

        base = wid * grows
        nch = jnp.where(wid == 31, last_extra // gch, grows // gch)

        def gbody(k, carry):
            r0 = base + k * gch
            pltpu.sync_copy(ex_hbm.at[pl.ds(r0 * 128, gch * 128)], gex_v)
            pltpu.sync_copy(idx_hbm.at[pl.ds(r0, gch)], gidx_v)
            for r in range(gch):
                for c2 in range(8):
                    i16 = gidx_v[r, pl.ds(c2 * 16, 16)]
                    x16 = gex_v[pl.ds(r * 128 + c2 * 16, 16)]
                    d16 = plsc.load_gather(denom_v, [i16])
                    out_v[r, pl.ds(c2 * 16, 16)] = x16 / (d16 + 1e-16)
            pltpu.sync_copy(out_v, out_hbm.at[pl.ds(r0, gch)])
            return carry
        lax.fori_loop(0, nch, gbody, 0)

    return body(ex, idx2, partials)


def kernel(x_i, x_j, edge_index, num_nodes, a):
    n = x_i.shape[0]
    w_flat = (a[:, 0, :_DIM] * a[:, 0, _DIM:]).reshape(_HEADS * _DIM)
    idx = edge_index[1] + (num_nodes - _NUM_NODES).astype(edge_index.dtype)
    idx2 = idx.reshape(n // 128, 128)
    ex, partials = _sc_dense_scatter(x_i.T, x_j.T, idx2, w_flat)
    out = _sc_gather_div(ex, idx2, partials)
    return out.reshape(n, 1)


# speedup vs baseline: 3.8450x; 1.2231x over previous
import functools

import jax
import jax.numpy as jnp
from jax import lax
from jax.experimental import pallas as pl
from jax.experimental.pallas import tpu as pltpu
from jax.experimental.pallas import tpu_sc as plsc

_HEADS = 4
_DIM = 32
_NUM_NODES = 10000
_SEG = _HEADS * _NUM_NODES
_SEG_PAD = 40960
_SLC = _SEG_PAD // 16

_CHUNK = 1024


def _mesh():
    return plsc.VectorSubcoreMesh(
        core_axis_name="c", subcore_axis_name="s", num_cores=2, num_subcores=16
    )


def _sc_dense_scatter(xt_i, xt_j, idx2, w_flat):
    n = xt_i.shape[1]
    rows = n // 128
    eph = n // _HEADS
    nchunks = n // _CHUNK
    base_trips = nchunks // 32
    extra = nchunks % 32

    quarter = 256

    @functools.partial(
        pl.kernel,
        out_type=(
            jax.ShapeDtypeStruct((n,), jnp.float32),
            jax.ShapeDtypeStruct((2 * _SEG_PAD,), jnp.float32),
        ),
        mesh=_mesh(),
        compiler_params=pltpu.CompilerParams(needs_layout_passes=False),
        scratch_types=[
            pltpu.VMEM((_DIM, quarter), jnp.float32),
            pltpu.VMEM((_DIM, quarter), jnp.float32),
            pltpu.VMEM((_DIM, quarter), jnp.float32),
            pltpu.VMEM((_DIM, quarter), jnp.float32),
            pltpu.VMEM((8, 128), jnp.int32),
            pltpu.VMEM((_CHUNK,), jnp.float32),
            pltpu.VMEM((_SLC,), jnp.float32),
            pltpu.VMEM((128,), jnp.float32),
            pltpu.VMEM_SHARED((_SEG_PAD,), jnp.float32),
            pltpu.SemaphoreType.DMA,
            pltpu.SemaphoreType.DMA,
            pltpu.SemaphoreType.DMA,
        ],
    )
    def body(xi_hbm, xj_hbm, idx_hbm, w_hbm, ex_hbm, part_hbm,
             xi_a, xj_a, xi_c, xj_c, sidx_v, ex_v, zbuf, wv, denom_sh,
             sem_a, sem_c, ssem):
        cid = lax.axis_index("c")
        sid = lax.axis_index("s")
        wid = cid * 16 + sid

        pltpu.sync_copy(w_hbm, wv)

        def zbody(i, carry):
            zbuf[pl.ds(i * 16, 16)] = jnp.zeros((16,), jnp.float32)
            return carry
        lax.fori_loop(0, _SLC // 16, zbody, 0)
        pltpu.sync_copy(zbuf, denom_sh.at[pl.ds(sid * _SLC, _SLC)])
        plsc.subcore_barrier()

        trips = jnp.where(wid < extra, base_trips + 1, base_trips)

        bufs = [(xi_a, xj_a, sem_a), (xi_c, xj_c, sem_c)]

        def fire_q(e0, q):
            eq = e0 + q * quarter
            bi, bj, sm = bufs[q % 2]
            return (pltpu.async_copy(xi_hbm.at[:, pl.ds(eq, quarter)],
                                     bi, sm),
                    pltpu.async_copy(xj_hbm.at[:, pl.ds(eq, quarter)],
                                     bj, sm))

        def chunk_body(k, carry):
            c = k * 32 + wid
            e0 = c * _CHUNK
            pltpu.sync_copy(idx_hbm.at[pl.ds(c * 8, 8)], sidx_v)
            pend = fire_q(e0, 0)
            for q in range(4):
                bi, bj, sm = bufs[q % 2]
                pend[0].wait()
                pend[1].wait()
                if q < 3:
                    pend = fire_q(e0, q + 1)
                head = (e0 + q * quarter) // eph
                wbc = [
                    plsc.load_gather(
                        wv, [jnp.full((16,), head * _DIM + d, jnp.int32)])
                    for d in range(_DIM)
                ]

                def group_body(g, carry2):
                    sl = pl.ds(g * 16, 16)
                    acc = bi[0, sl] * bj[0, sl] * wbc[0]
                    for d in range(1, _DIM):
                        acc = acc + bi[d, sl] * bj[d, sl] * wbc[d]
                    ex_v[pl.ds(q * quarter + g * 16, 16)] = jnp.exp(acc)
                    return carry2
                lax.fori_loop(0, quarter // 16, group_body, 0)

            pltpu.sync_copy(ex_v, ex_hbm.at[pl.ds(e0, _CHUNK)])
            copies = []
            for j in range(8):
                copies.append(pltpu.async_copy(
                    ex_v.at[pl.ds(j * 128, 128)],
                    denom_sh.at[sidx_v.at[j]], ssem, add=True))
            for cp in copies:
                cp.wait()
            return carry
        lax.fori_loop(0, trips, chunk_body, 0)
        plsc.subcore_barrier()

        pltpu.sync_copy(
            denom_sh.at[pl.ds(sid * _SLC, _SLC)],
            part_hbm.at[pl.ds(cid * _SEG_PAD + sid * _SLC, _SLC)],
        )

    return body(xt_i, xt_j, idx2, w_flat)


def _sc_gather_div(ex, idx2, partials):
    rows = idx2.shape[0]
    grows = (rows // 32) & ~7
    gch = 8
    last_extra = rows - 31 * grows

    @functools.partial(
        pl.kernel,
        out_type=jax.ShapeDtypeStruct((rows, 128), jnp.float32),
        mesh=_mesh(),
        compiler_params=pltpu.CompilerParams(needs_layout_passes=False),
        scratch_types=[
            pltpu.VMEM((_SEG_PAD,), jnp.float32),
            pltpu.VMEM((_SLC,), jnp.float32),
            pltpu.VMEM((gch * 128,), jnp.float32),
            pltpu.VMEM((gch, 128), jnp.int32),
            pltpu.VMEM((gch, 128), jnp.float32),
        ],
    )
    def body(ex_hbm, idx_hbm, part_hbm, out_hbm,
             denom_v, tmp_v, gex_v, gidx_v, out_v):
        cid = lax.axis_index("c")
        sid = lax.axis_index("s")
        wid = cid * 16 + sid

        pltpu.sync_copy(part_hbm.at[pl.ds(0, _SEG_PAD)], denom_v)

        def merge_chunk(j, carry):
            pltpu.sync_copy(
                part_hbm.at[pl.ds(_SEG_PAD + j * _SLC, _SLC)], tmp_v)

            def madd(i, c2):
                off = i * 16
                denom_v[pl.ds(j * _SLC + off, 16)] = (
                    denom_v[pl.ds(j * _SLC + off, 16)]
                    + tmp_v[pl.ds(off, 16)])
                return c2
            lax.fori_loop(0, _SLC // 16, madd, 0)
            return carry
        lax.fori_loop(0, 16, merge_chunk, 0)

        base = wid * grows
        nch = jnp.where(wid == 31, last_extra // gch, grows // gch)

        def gbody(k, carry):
            r0 = base + k * gch
            pltpu.sync_copy(ex_hbm.at[pl.ds(r0 * 128, gch * 128)], gex_v)
            pltpu.sync_copy(idx_hbm.at[pl.ds(r0, gch)], gidx_v)
            for r in range(gch):
                for c2 in range(8):
                    i16 = gidx_v[r, pl.ds(c2 * 16, 16)]
                    x16 = gex_v[pl.ds(r * 128 + c2 * 16, 16)]
                    d16 = plsc.load_gather(denom_v, [i16])
                    out_v[r, pl.ds(c2 * 16, 16)] = x16 / (d16 + 1e-16)
            pltpu.sync_copy(out_v, out_hbm.at[pl.ds(r0, gch)])
            return carry
        lax.fori_loop(0, nch, gbody, 0)

    return body(ex, idx2, partials)


def kernel(x_i, x_j, edge_index, num_nodes, a):
    n = x_i.shape[0]
    w_flat = (a[:, 0, :_DIM] * a[:, 0, _DIM:]).reshape(_HEADS * _DIM)
    idx = edge_index[1] + (num_nodes - _NUM_NODES).astype(edge_index.dtype)
    idx2 = idx.reshape(n // 128, 128)
    ex, partials = _sc_dense_scatter(x_i.T, x_j.T, idx2, w_flat)
    out = _sc_gather_div(ex, idx2, partials)
    return out.reshape(n, 1)


# speedup vs baseline: 4.6474x; 1.2087x over previous
import functools

import jax
import jax.numpy as jnp
from jax import lax
from jax.experimental import pallas as pl
from jax.experimental.pallas import tpu as pltpu
from jax.experimental.pallas import tpu_sc as plsc

_HEADS = 4
_DIM = 32
_NUM_NODES = 10000
_SEG = _HEADS * _NUM_NODES
_SEG_PAD = 40960
_SLC = _SEG_PAD // 16

_CHUNK = 1024


def _mesh():
    return plsc.VectorSubcoreMesh(
        core_axis_name="c", subcore_axis_name="s", num_cores=2, num_subcores=16
    )


def _sc_dense_scatter(xt_i, xt_j, idx2, w_flat):
    n = xt_i.shape[1]
    rows = n // 128
    eph = n // _HEADS
    nchunks = n // _CHUNK
    base_trips = nchunks // 32
    extra = nchunks % 32

    quarter = 256

    @functools.partial(
        pl.kernel,
        out_type=(
            jax.ShapeDtypeStruct((n,), jnp.float32),
            jax.ShapeDtypeStruct((2 * _SEG_PAD,), jnp.float32),
        ),
        mesh=_mesh(),
        compiler_params=pltpu.CompilerParams(needs_layout_passes=False),
        scratch_types=[
            pltpu.VMEM((_DIM, quarter), jnp.float32),
            pltpu.VMEM((_DIM, quarter), jnp.float32),
            pltpu.VMEM((_DIM, quarter), jnp.float32),
            pltpu.VMEM((_DIM, quarter), jnp.float32),
            pltpu.VMEM((_DIM, quarter), jnp.float32),
            pltpu.VMEM((_DIM, quarter), jnp.float32),
            pltpu.VMEM((8, 128), jnp.int32),
            pltpu.VMEM((_CHUNK,), jnp.float32),
            pltpu.VMEM((_SLC,), jnp.float32),
            pltpu.VMEM((128,), jnp.float32),
            pltpu.VMEM_SHARED((_SEG_PAD,), jnp.float32),
            pltpu.SemaphoreType.DMA,
            pltpu.SemaphoreType.DMA,
            pltpu.SemaphoreType.DMA,
            pltpu.SemaphoreType.DMA,
        ],
    )
    def body(xi_hbm, xj_hbm, idx_hbm, w_hbm, ex_hbm, part_hbm,
             xi_a, xj_a, xi_c, xj_c, xi_e, xj_e, sidx_v, ex_v, zbuf, wv,
             denom_sh, sem_a, sem_c, sem_e, ssem):
        cid = lax.axis_index("c")
        sid = lax.axis_index("s")
        wid = cid * 16 + sid

        pltpu.sync_copy(w_hbm, wv)

        def zbody(i, carry):
            zbuf[pl.ds(i * 16, 16)] = jnp.zeros((16,), jnp.float32)
            return carry
        lax.fori_loop(0, _SLC // 16, zbody, 0)
        pltpu.sync_copy(zbuf, denom_sh.at[pl.ds(sid * _SLC, _SLC)])
        plsc.subcore_barrier()

        trips = jnp.where(wid < extra, base_trips + 1, base_trips)

        bufs = [(xi_a, xj_a, sem_a), (xi_c, xj_c, sem_c), (xi_e, xj_e, sem_e)]

        def fire_q(e0, q):
            eq = e0 + q * quarter
            bi, bj, sm = bufs[q % 3]
            return (pltpu.async_copy(xi_hbm.at[:, pl.ds(eq, quarter)],
                                     bi, sm),
                    pltpu.async_copy(xj_hbm.at[:, pl.ds(eq, quarter)],
                                     bj, sm))

        def chunk_body(k, carry):
            c = k * 32 + wid
            e0 = c * _CHUNK
            pltpu.sync_copy(idx_hbm.at[pl.ds(c * 8, 8)], sidx_v)
            pend = [fire_q(e0, 0), fire_q(e0, 1)]
            for q in range(4):
                bi, bj, sm = bufs[q % 3]
                pend[q][0].wait()
                pend[q][1].wait()
                if q < 2:
                    pend.append(fire_q(e0, q + 2))
                head = (e0 + q * quarter) // eph
                wbc = [
                    plsc.load_gather(
                        wv, [jnp.full((16,), head * _DIM + d, jnp.int32)])
                    for d in range(_DIM)
                ]

                def group_body(g, carry2):
                    sl = pl.ds(g * 16, 16)
                    acc = bi[0, sl] * bj[0, sl] * wbc[0]
                    for d in range(1, _DIM):
                        acc = acc + bi[d, sl] * bj[d, sl] * wbc[d]
                    ex_v[pl.ds(q * quarter + g * 16, 16)] = jnp.exp(acc)
                    return carry2
                lax.fori_loop(0, quarter // 16, group_body, 0)

            pltpu.sync_copy(ex_v, ex_hbm.at[pl.ds(e0, _CHUNK)])
            copies = []
            for j in range(8):
                copies.append(pltpu.async_copy(
                    ex_v.at[pl.ds(j * 128, 128)],
                    denom_sh.at[sidx_v.at[j]], ssem, add=True))
            for cp in copies:
                cp.wait()
            return carry
        lax.fori_loop(0, trips, chunk_body, 0)
        plsc.subcore_barrier()

        pltpu.sync_copy(
            denom_sh.at[pl.ds(sid * _SLC, _SLC)],
            part_hbm.at[pl.ds(cid * _SEG_PAD + sid * _SLC, _SLC)],
        )

    return body(xt_i, xt_j, idx2, w_flat)


def _sc_gather_div(ex, idx2, partials):
    rows = idx2.shape[0]
    grows = 320
    gch = 16

    @functools.partial(
        pl.kernel,
        out_type=jax.ShapeDtypeStruct((rows, 128), jnp.float32),
        mesh=_mesh(),
        compiler_params=pltpu.CompilerParams(needs_layout_passes=False),
        scratch_types=[
            pltpu.VMEM((_SEG_PAD,), jnp.float32),
            pltpu.VMEM((_SLC,), jnp.float32),
            pltpu.VMEM((_SLC,), jnp.float32),
            pltpu.VMEM((8 * 128,), jnp.float32),
            pltpu.VMEM((8, 128), jnp.int32),
            pltpu.VMEM((8 * 128,), jnp.float32),
            pltpu.VMEM((8, 128), jnp.int32),
            pltpu.VMEM((gch, 128), jnp.float32),
            pltpu.SemaphoreType.DMA,
            pltpu.SemaphoreType.DMA,
        ],
    )
    def body(ex_hbm, idx_hbm, part_hbm, out_hbm,
             denom_v, tmp_a, tmp_b, gex_a, gidx_a, gex_b, gidx_b, out_v,
             sem_a, sem_b):
        cid = lax.axis_index("c")
        sid = lax.axis_index("s")
        wid = cid * 16 + sid

        pltpu.sync_copy(part_hbm.at[pl.ds(0, _SEG_PAD)], denom_v)
        tbufs = [(tmp_a, sem_a), (tmp_b, sem_b)]
        pend = pltpu.async_copy(part_hbm.at[pl.ds(_SEG_PAD, _SLC)],
                                tmp_a, sem_a)
        for j in range(16):
            tv, _ = tbufs[j % 2]
            pend.wait()
            if j < 15:
                nv, nsm = tbufs[(j + 1) % 2]
                pend = pltpu.async_copy(
                    part_hbm.at[pl.ds(_SEG_PAD + (j + 1) * _SLC, _SLC)],
                    nv, nsm)

            def madd(i, c2):
                off = i * 16
                denom_v[pl.ds(j * _SLC + off, 16)] = (
                    denom_v[pl.ds(j * _SLC + off, 16)]
                    + tv[pl.ds(off, 16)])
                return c2
            lax.fori_loop(0, _SLC // 16, madd, 0)

        base = wid * grows
        nch = jnp.where(wid == 31, (rows - 31 * grows) // gch, grows // gch)
        gbufs = [(gex_a, gidx_a, sem_a), (gex_b, gidx_b, sem_b)]

        def fire_s(r0, s):
            gx, gi, sm = gbufs[s % 2]
            rs = r0 + s * 8
            return (pltpu.async_copy(ex_hbm.at[pl.ds(rs * 128, 1024)],
                                     gx, sm),
                    pltpu.async_copy(idx_hbm.at[pl.ds(rs, 8)], gi, sm))

        def gbody(k, carry):
            r0 = base + k * gch
            pend2 = fire_s(r0, 0)
            for s in range(2):
                gx, gi, sm = gbufs[s % 2]
                pend2[0].wait()
                pend2[1].wait()
                if s == 0:
                    pend2 = fire_s(r0, 1)
                for r in range(8):
                    for c2 in range(8):
                        i16 = gi[r, pl.ds(c2 * 16, 16)]
                        x16 = gx[pl.ds(r * 128 + c2 * 16, 16)]
                        d16 = plsc.load_gather(denom_v, [i16])
                        out_v[s * 8 + r, pl.ds(c2 * 16, 16)] = (
                            x16 / (d16 + 1e-16))
            pltpu.sync_copy(out_v, out_hbm.at[pl.ds(r0, gch)])
            return carry
        lax.fori_loop(0, nch, gbody, 0)

    return body(ex, idx2, partials)


def kernel(x_i, x_j, edge_index, num_nodes, a):
    n = x_i.shape[0]
    w_flat = (a[:, 0, :_DIM] * a[:, 0, _DIM:]).reshape(_HEADS * _DIM)
    idx = edge_index[1] + (num_nodes - _NUM_NODES).astype(edge_index.dtype)
    idx2 = idx.reshape(n // 128, 128)
    ex, partials = _sc_dense_scatter(x_i.T, x_j.T, idx2, w_flat)
    out = _sc_gather_div(ex, idx2, partials)
    return out.reshape(n, 1)
